# baseline (speedup 1.0000x reference)
import dataclasses
import functools

import jax
import jax.numpy as jnp
from jax import lax
from jax.experimental import pallas as pl
from jax.experimental.pallas import tpu as pltpu
from jax.experimental.pallas import tpu_sc as plsc

B = 16384
D = 32
NC = 2
NS = 16
NW = NC * NS
BPW = B // NW
L = 16
CH = 128


def _fm_body(user_hbm, item_hbm, ue_hbm, ie_hbm, ub_hbm, ib_hbm, gb_hbm,
             out_hbm,
             uidx_v, iidx_v, udiv_v, idiv_v, urows_v, irows_v,
             ubias_v, ibias_v, gb_v, out_v,
             sem_u, sem_i, sem_ub, sem_ib):
    c = lax.axis_index("c")
    s = lax.axis_index("s")
    wid = s * NC + c
    base = wid * BPW

    pltpu.sync_copy(user_hbm.at[pl.ds(base, BPW)], uidx_v)
    pltpu.sync_copy(item_hbm.at[pl.ds(base, BPW)], iidx_v)
    pltpu.sync_copy(gb_hbm, gb_v)

    @pl.loop(0, BPW, step=L)
    def _(r0):
        sl = pl.ds(r0, L)
        udiv_v[sl] = lax.shift_right_logical(uidx_v[sl], 4)
        idiv_v[sl] = lax.shift_right_logical(iidx_v[sl], 4)

    copies = []
    for k in range(BPW // CH):
        sl = pl.ds(k * CH, CH)
        copies.append(pltpu.async_copy(
            ue_hbm.at[uidx_v.at[sl]], urows_v.at[sl], sem_u))
        copies.append(pltpu.async_copy(
            ie_hbm.at[iidx_v.at[sl]], irows_v.at[sl], sem_i))
        copies.append(pltpu.async_copy(
            ub_hbm.at[udiv_v.at[sl]], ubias_v.at[sl], sem_ub))
        copies.append(pltpu.async_copy(
            ib_hbm.at[idiv_v.at[sl]], ibias_v.at[sl], sem_ib))
    for cp in copies:
        cp.wait()

    lane = lax.iota(jnp.int32, L)
    low4 = jnp.full((L,), 15, jnp.int32)
    gb = gb_v[...]

    @pl.loop(0, BPW, step=L)
    def _(r0):
        rows = lane + r0
        acc = jnp.zeros((L,), jnp.float32)
        for d in range(D):
            col = jnp.full((L,), d, jnp.int32)
            ud = plsc.load_gather(urows_v, [rows, col])
            vd = plsc.load_gather(irows_v, [rows, col])
            acc = acc + ud * vd
        ubv = plsc.load_gather(ubias_v, [rows, uidx_v[pl.ds(r0, L)] & low4])
        ibv = plsc.load_gather(ibias_v, [rows, iidx_v[pl.ds(r0, L)] & low4])
        out_v[pl.ds(r0, L)] = acc + ubv + ibv + gb

    pltpu.sync_copy(out_v, out_hbm.at[pl.ds(base, BPW)])


@jax.jit
def _fm(user, item, user_embed, item_embed, user_bias, item_bias, global_bias):
    cp = pltpu.CompilerParams(use_tc_tiling_on_sc=False)
    if "needs_layout_passes" in pltpu.CompilerParams.__dataclass_fields__:
        cp = dataclasses.replace(cp, needs_layout_passes=False)
    run = pl.kernel(
        _fm_body,
        out_type=jax.ShapeDtypeStruct((B,), jnp.float32),
        mesh=plsc.VectorSubcoreMesh(core_axis_name="c", subcore_axis_name="s"),
        compiler_params=cp,
        scratch_types=[
            pltpu.VMEM((BPW,), jnp.int32),
            pltpu.VMEM((BPW,), jnp.int32),
            pltpu.VMEM((BPW,), jnp.int32),
            pltpu.VMEM((BPW,), jnp.int32),
            pltpu.VMEM((BPW, D), jnp.float32),
            pltpu.VMEM((BPW, D), jnp.float32),
            pltpu.VMEM((BPW, L), jnp.float32),
            pltpu.VMEM((BPW, L), jnp.float32),
            pltpu.VMEM((L,), jnp.float32),
            pltpu.VMEM((BPW,), jnp.float32),
            pltpu.SemaphoreType.DMA,
            pltpu.SemaphoreType.DMA,
            pltpu.SemaphoreType.DMA,
            pltpu.SemaphoreType.DMA,
        ],
    )
    return run(user, item, user_embed, item_embed, user_bias, item_bias,
               global_bias)


def kernel(user, item, user_embed, item_embed, user_bias, item_bias,
           global_bias):
    return _fm(user.astype(jnp.int32), item.astype(jnp.int32),
               user_embed, item_embed,
               user_bias.reshape(-1, L), item_bias.reshape(-1, L),
               jnp.broadcast_to(global_bias, (L,)))


# speedup vs baseline: 1.0007x; 1.0007x over previous
import dataclasses
import functools

import jax
import jax.numpy as jnp
from jax import lax
from jax.experimental import pallas as pl
from jax.experimental.pallas import tpu as pltpu
from jax.experimental.pallas import tpu_sc as plsc

B = 16384
D = 32
NC = 2
NS = 16
NW = NC * NS
BPW = B // NW
L = 16
CH = 128


def _fm_body(user_hbm, item_hbm, ue_hbm, ie_hbm, ub_hbm, ib_hbm, gb_hbm,
             out_hbm,
             uidx_v, iidx_v, urows_v, irows_v, ubias_v, ibias_v, gb_v, out_v,
             sem_u, sem_i, sem_ub, sem_ib):
    c = lax.axis_index("c")
    s = lax.axis_index("s")
    wid = s * NC + c
    base = wid * BPW

    pltpu.sync_copy(user_hbm.at[pl.ds(base, BPW)], uidx_v)
    pltpu.sync_copy(item_hbm.at[pl.ds(base, BPW)], iidx_v)
    pltpu.sync_copy(gb_hbm, gb_v)

    copies = []
    for k in range(BPW // CH):
        sl = pl.ds(k * CH, CH)
        copies.append(pltpu.async_copy(
            ue_hbm.at[uidx_v.at[sl]], urows_v.at[sl], sem_u))
        copies.append(pltpu.async_copy(
            ie_hbm.at[iidx_v.at[sl]], irows_v.at[sl], sem_i))
        copies.append(pltpu.async_copy(
            ub_hbm.at[uidx_v.at[sl]], ubias_v.at[sl], sem_ub))
        copies.append(pltpu.async_copy(
            ib_hbm.at[iidx_v.at[sl]], ibias_v.at[sl], sem_ib))
    for cp in copies:
        cp.wait()

    lane = lax.iota(jnp.int32, L)
    gb = gb_v[...]

    @pl.loop(0, BPW, step=L)
    def _(r0):
        rows = lane + r0
        acc = jnp.zeros((L,), jnp.float32)
        for d in range(D):
            col = jnp.full((L,), d, jnp.int32)
            ud = plsc.load_gather(urows_v, [rows, col])
            vd = plsc.load_gather(irows_v, [rows, col])
            acc = acc + ud * vd
        sl = pl.ds(r0, L)
        out_v[sl] = acc + ubias_v[sl] + ibias_v[sl] + gb

    pltpu.sync_copy(out_v, out_hbm.at[pl.ds(base, BPW)])


@jax.jit
def _fm(user, item, user_embed, item_embed, user_bias, item_bias, global_bias):
    cp = pltpu.CompilerParams(use_tc_tiling_on_sc=False)
    if "needs_layout_passes" in pltpu.CompilerParams.__dataclass_fields__:
        cp = dataclasses.replace(cp, needs_layout_passes=False)
    run = pl.kernel(
        _fm_body,
        out_type=jax.ShapeDtypeStruct((B,), jnp.float32),
        mesh=plsc.VectorSubcoreMesh(core_axis_name="c", subcore_axis_name="s"),
        compiler_params=cp,
        scratch_types=[
            pltpu.VMEM((BPW,), jnp.int32),
            pltpu.VMEM((BPW,), jnp.int32),
            pltpu.VMEM((BPW, D), jnp.float32),
            pltpu.VMEM((BPW, D), jnp.float32),
            pltpu.VMEM((BPW,), jnp.float32),
            pltpu.VMEM((BPW,), jnp.float32),
            pltpu.VMEM((L,), jnp.float32),
            pltpu.VMEM((BPW,), jnp.float32),
            pltpu.SemaphoreType.DMA,
            pltpu.SemaphoreType.DMA,
            pltpu.SemaphoreType.DMA,
            pltpu.SemaphoreType.DMA,
        ],
    )
    return run(user, item, user_embed, item_embed, user_bias, item_bias,
               global_bias)


def kernel(user, item, user_embed, item_embed, user_bias, item_bias,
           global_bias):
    return _fm(user.astype(jnp.int32), item.astype(jnp.int32),
               user_embed, item_embed,
               user_bias.reshape(-1), item_bias.reshape(-1),
               jnp.broadcast_to(global_bias, (L,)))
